# baseline (speedup 1.0000x reference)
import functools

import jax
import jax.numpy as jnp
from jax import lax
from jax.experimental import pallas as pl
from jax.experimental.pallas import tpu as pltpu
from jax.experimental.pallas import tpu_sc as plsc

_B, _N, _M = 16, 4096, 1024
_C1, _C2 = 256, 512
_O1, _O2 = 512, 256
_NTOT = _B * _N
_NBLK_A = 256
_NBLK = 512

_NW = 32
_PPW = _NTOT // _NW
_CS = 32
_JW = _O1 // 16


def _gt_body(kf_ref, w1a_ref, gt_ref):
    gt_ref[...] = lax.dot_general(
        kf_ref[...], w1a_ref[...], (((0,), (1,)), ((), ())),
        preferred_element_type=jnp.float32)


def _knn_body(ut_ref, kt_ref, idx_ref, w_ref):
    u = ut_ref[...]
    kv = kt_ref[...]
    uu = jnp.sum(u * u, axis=0)[None, :]
    kk = jnp.sum(kv * kv, axis=0)[:, None]
    cross = lax.dot_general(kv, u, (((0,), (0,)), ((), ())),
                            preferred_element_type=jnp.float32)
    d = jnp.maximum(kk + uu - 2.0 * cross, 0.0)
    iota = lax.broadcasted_iota(jnp.int32, (_M, _NBLK_A), 0)
    recips, imins = [], []
    for _ in range(3):
        vmin = jnp.min(d, axis=0, keepdims=True)
        imin = jnp.min(jnp.where(d == vmin, iota, _M), axis=0, keepdims=True)
        recips.append(1.0 / (jnp.sqrt(vmin) + 1e-8))
        imins.append(imin)
        d = jnp.where(iota == imin, jnp.float32(jnp.inf), d)
    norm = recips[0] + recips[1] + recips[2]
    gbase = pl.program_id(0) * _M
    idx_ref[...] = jnp.concatenate(
        [imins[0] + gbase, imins[1] + gbase, imins[2] + gbase,
         jnp.zeros((5, _NBLK_A), jnp.int32)], axis=0)
    w_ref[...] = jnp.concatenate(
        [recips[0] / norm, recips[1] / norm, recips[2] / norm,
         jnp.zeros((5, _NBLK_A), jnp.float32)], axis=0)


def _interp_body(idx_hbm, w_hbm, gt_hbm, out_hbm,
                 i0_v, i1_v, i2_v, w0_v, w1_v, w2_v,
                 r0_v, r1_v, r2_v, out_v, s0, s1, s2):
    wid = lax.axis_index("s") * 2 + lax.axis_index("c")
    base0 = wid * _PPW

    def chunk(ci, carry):
        base = base0 + ci * _CS
        pltpu.sync_copy(idx_hbm.at[0, pl.ds(base, _CS)], i0_v)
        pltpu.sync_copy(idx_hbm.at[1, pl.ds(base, _CS)], i1_v)
        pltpu.sync_copy(idx_hbm.at[2, pl.ds(base, _CS)], i2_v)
        pltpu.sync_copy(w_hbm.at[0, pl.ds(base, _CS)], w0_v.at[pl.ds(0, _CS)])
        pltpu.sync_copy(w_hbm.at[1, pl.ds(base, _CS)], w1_v.at[pl.ds(0, _CS)])
        pltpu.sync_copy(w_hbm.at[2, pl.ds(base, _CS)], w2_v.at[pl.ds(0, _CS)])
        c0 = pltpu.async_copy(gt_hbm.at[i0_v], r0_v, s0)
        c1 = pltpu.async_copy(gt_hbm.at[i1_v], r1_v, s1)
        c2 = pltpu.async_copy(gt_hbm.at[i2_v], r2_v, s2)
        c0.wait()
        c1.wait()
        c2.wait()

        def point(i, c):
            a0 = jnp.full((16,), w0_v[pl.ds(i, 16)][0], jnp.float32)
            a1 = jnp.full((16,), w1_v[pl.ds(i, 16)][0], jnp.float32)
            a2 = jnp.full((16,), w2_v[pl.ds(i, 16)][0], jnp.float32)
            for j in range(_JW):
                sl = pl.ds(j * 16, 16)
                out_v[i, sl] = (r0_v[i, sl] * a0 + r1_v[i, sl] * a1
                                + r2_v[i, sl] * a2)
            return c

        lax.fori_loop(0, _CS, point, 0)
        pltpu.sync_copy(out_v, out_hbm.at[pl.ds(base, _CS)])
        return carry

    lax.fori_loop(0, _PPW // _CS, chunk, 0)


def _e_body(interp_ref, uf_ref, w1b_ref, b1_ref, y1_ref, st_ref):
    y = lax.dot_general(uf_ref[...], w1b_ref[...], (((0,), (1,)), ((), ())),
                        preferred_element_type=jnp.float32)
    y = y + interp_ref[...] + b1_ref[...]
    y1_ref[...] = y
    s = jnp.sum(y, axis=0, keepdims=True)
    s2 = jnp.sum(y * y, axis=0, keepdims=True)

    @pl.when(jnp.logical_and(pl.program_id(0) == 0, pl.program_id(1) == 0))
    def _():
        st_ref[...] = jnp.zeros_like(st_ref)

    st_ref[...] = st_ref[...] + jnp.concatenate(
        [s, s2, jnp.zeros((6, _O1), jnp.float32)], axis=0)


def _f_body(y1_ref, st1_ref, g1_ref, bt1_ref, w2_ref, b2_ref, y2_ref, st_ref):
    st = st1_ref[...]
    mean = st[0:1, :] / _NTOT
    var = st[1:2, :] / _NTOT - mean * mean
    inv = lax.rsqrt(var + 1e-5) * g1_ref[...]
    h = jnp.maximum((y1_ref[...] - mean) * inv + bt1_ref[...], 0.0)
    y2 = lax.dot_general(w2_ref[...], h, (((1,), (1,)), ((), ())),
                         preferred_element_type=jnp.float32)
    y2 = y2 + b2_ref[...]
    y2_ref[...] = y2
    s = jnp.sum(y2, axis=1, keepdims=True)
    s2 = jnp.sum(y2 * y2, axis=1, keepdims=True)

    @pl.when(pl.program_id(0) == 0)
    def _():
        st_ref[...] = jnp.zeros_like(st_ref)

    st_ref[...] = st_ref[...] + jnp.concatenate(
        [s, s2, jnp.zeros((_O2, 6), jnp.float32)], axis=1)


def _g_body(y2_ref, st2_ref, g2_ref, bt2_ref, out_ref):
    st = st2_ref[...]
    mean = st[:, 0:1] / _NTOT
    var = st[:, 1:2] / _NTOT - mean * mean
    inv = lax.rsqrt(var + 1e-5) * g2_ref[...]
    out_ref[...] = jnp.maximum((y2_ref[...] - mean) * inv + bt2_ref[...], 0.0)


def kernel(unknown, known, unknow_feats, known_feats,
           W1, b1, g1, bt1, W2, b2, g2, bt2):
    f32 = jnp.float32
    ut8 = jnp.concatenate(
        [jnp.transpose(unknown, (0, 2, 1)), jnp.zeros((_B, 5, _N), f32)], axis=1)
    kt8 = jnp.concatenate(
        [jnp.transpose(known, (0, 2, 1)), jnp.zeros((_B, 5, _M), f32)], axis=1)
    W1a = W1[:, :_C2]
    W1b = W1[:, _C2:]
    b1r = b1.reshape(1, _O1)
    g1r = g1.reshape(1, _O1)
    bt1r = bt1.reshape(1, _O1)
    b2r = b2.reshape(_O2, 1)
    g2r = g2.reshape(_O2, 1)
    bt2r = bt2.reshape(_O2, 1)

    gt = pl.pallas_call(
        _gt_body,
        grid=(_B,),
        in_specs=[
            pl.BlockSpec((None, _C2, _M), lambda b: (b, 0, 0)),
            pl.BlockSpec((_O1, _C2), lambda b: (0, 0)),
        ],
        out_specs=pl.BlockSpec((_M, _O1), lambda b: (b, 0)),
        out_shape=jax.ShapeDtypeStruct((_B * _M, _O1), f32),
    )(known_feats, W1a)

    nja = _N // _NBLK_A
    idx8, w8 = pl.pallas_call(
        _knn_body,
        grid=(_B, nja),
        in_specs=[
            pl.BlockSpec((None, 8, _NBLK_A), lambda b, j: (b, 0, j)),
            pl.BlockSpec((None, 8, _M), lambda b, j: (b, 0, 0)),
        ],
        out_specs=[
            pl.BlockSpec((8, _NBLK_A), lambda b, j: (0, b * nja + j)),
            pl.BlockSpec((8, _NBLK_A), lambda b, j: (0, b * nja + j)),
        ],
        out_shape=[
            jax.ShapeDtypeStruct((8, _NTOT), jnp.int32),
            jax.ShapeDtypeStruct((8, _NTOT), f32),
        ],
    )(ut8, kt8)

    interp = pl.kernel(
        _interp_body,
        out_type=jax.ShapeDtypeStruct((_NTOT, _O1), f32),
        mesh=plsc.VectorSubcoreMesh(core_axis_name="c", subcore_axis_name="s"),
        scratch_types=[
            pltpu.VMEM((_CS,), jnp.int32),
            pltpu.VMEM((_CS,), jnp.int32),
            pltpu.VMEM((_CS,), jnp.int32),
            pltpu.VMEM((_CS + 16,), f32),
            pltpu.VMEM((_CS + 16,), f32),
            pltpu.VMEM((_CS + 16,), f32),
            pltpu.VMEM((_CS, _O1), f32),
            pltpu.VMEM((_CS, _O1), f32),
            pltpu.VMEM((_CS, _O1), f32),
            pltpu.VMEM((_CS, _O1), f32),
            pltpu.SemaphoreType.DMA,
            pltpu.SemaphoreType.DMA,
            pltpu.SemaphoreType.DMA,
        ],
    )(idx8, w8, gt)

    nj = _N // _NBLK
    y1, st1 = pl.pallas_call(
        _e_body,
        grid=(_B, nj),
        in_specs=[
            pl.BlockSpec((_NBLK, _O1), lambda b, j: (b * nj + j, 0)),
            pl.BlockSpec((None, _C1, _NBLK), lambda b, j: (b, 0, j)),
            pl.BlockSpec((_O1, _C1), lambda b, j: (0, 0)),
            pl.BlockSpec((1, _O1), lambda b, j: (0, 0)),
        ],
        out_specs=[
            pl.BlockSpec((_NBLK, _O1), lambda b, j: (b * nj + j, 0)),
            pl.BlockSpec((8, _O1), lambda b, j: (0, 0)),
        ],
        out_shape=[
            jax.ShapeDtypeStruct((_NTOT, _O1), f32),
            jax.ShapeDtypeStruct((8, _O1), f32),
        ],
    )(interp, unknow_feats, W1b, b1r)

    nt = _NTOT // _NBLK
    y2, st2 = pl.pallas_call(
        _f_body,
        grid=(nt,),
        in_specs=[
            pl.BlockSpec((_NBLK, _O1), lambda t: (t, 0)),
            pl.BlockSpec((8, _O1), lambda t: (0, 0)),
            pl.BlockSpec((1, _O1), lambda t: (0, 0)),
            pl.BlockSpec((1, _O1), lambda t: (0, 0)),
            pl.BlockSpec((_O2, _O1), lambda t: (0, 0)),
            pl.BlockSpec((_O2, 1), lambda t: (0, 0)),
        ],
        out_specs=[
            pl.BlockSpec((None, _O2, _NBLK), lambda t: (t // nj, 0, t % nj)),
            pl.BlockSpec((_O2, 8), lambda t: (0, 0)),
        ],
        out_shape=[
            jax.ShapeDtypeStruct((_B, _O2, _N), f32),
            jax.ShapeDtypeStruct((_O2, 8), f32),
        ],
    )(y1, st1, g1r, bt1r, W2, b2r)

    out = pl.pallas_call(
        _g_body,
        grid=(_B, nj),
        in_specs=[
            pl.BlockSpec((None, _O2, _NBLK), lambda b, j: (b, 0, j)),
            pl.BlockSpec((_O2, 8), lambda b, j: (0, 0)),
            pl.BlockSpec((_O2, 1), lambda b, j: (0, 0)),
            pl.BlockSpec((_O2, 1), lambda b, j: (0, 0)),
        ],
        out_specs=pl.BlockSpec((None, _O2, _NBLK), lambda b, j: (b, 0, j)),
        out_shape=jax.ShapeDtypeStruct((_B, _O2, _N), f32),
    )(y2, st2, g2r, bt2r)
    return out


# speedup vs baseline: 1.2860x; 1.2860x over previous
import functools

import jax
import jax.numpy as jnp
from jax import lax
from jax.experimental import pallas as pl
from jax.experimental.pallas import tpu as pltpu
from jax.experimental.pallas import tpu_sc as plsc

_B, _N, _M = 16, 4096, 1024
_C1, _C2 = 256, 512
_O1, _O2 = 512, 256
_NTOT = _B * _N
_NBLK_A = 256
_NBLK = 512

_NW = 32
_PPW = _NTOT // _NW
_CS = 32
_NCH = _PPW // _CS
_JW = _O1 // 16


def _gt_body(kf_ref, w1a_ref, gt_ref):
    gt_ref[...] = lax.dot_general(
        kf_ref[...], w1a_ref[...], (((0,), (1,)), ((), ())),
        preferred_element_type=jnp.float32)


def _knn_body(ut_ref, kt_ref, meta_ref):
    u = ut_ref[...]
    kv = kt_ref[...]
    uu = jnp.sum(u * u, axis=0)[None, :]
    kk = jnp.sum(kv * kv, axis=0)[:, None]
    cross = lax.dot_general(kv, u, (((0,), (0,)), ((), ())),
                            preferred_element_type=jnp.float32)
    d = jnp.maximum(kk + uu - 2.0 * cross, 0.0)
    iota = lax.broadcasted_iota(jnp.int32, (_M, _NBLK_A), 0)
    recips, imins = [], []
    for _ in range(3):
        vmin = jnp.min(d, axis=0, keepdims=True)
        imin = jnp.min(jnp.where(d == vmin, iota, _M), axis=0, keepdims=True)
        recips.append(1.0 / (jnp.sqrt(vmin) + 1e-8))
        imins.append(imin)
        d = jnp.where(iota == imin, jnp.float32(jnp.inf), d)
    norm = recips[0] + recips[1] + recips[2]
    gbase = pl.program_id(0) * _M

    bits = lambda x: lax.bitcast_convert_type(x, jnp.int32)
    meta_ref[...] = jnp.concatenate(
        [imins[0] + gbase, imins[1] + gbase, imins[2] + gbase,
         bits(recips[0] / norm), bits(recips[1] / norm),
         bits(recips[2] / norm),
         jnp.zeros((2, _NBLK_A), jnp.int32)], axis=0)


def _interp_body(meta_hbm, gt_hbm, out_hbm,
                 meta0_v, meta1_v, r00, r01, r02, r10, r11, r12, out_v,
                 s00, s01, s02, s10, s11, s12):
    wid = lax.axis_index("s") * 2 + lax.axis_index("c")
    base0 = wid * _PPW
    metas = (meta0_v, meta1_v)
    rows = ((r00, r01, r02), (r10, r11, r12))
    sems = ((s00, s01, s02), (s10, s11, s12))

    def fire(ci, p):
        base = base0 + ci * _CS
        pltpu.sync_copy(meta_hbm.at[pl.ds(0, 8), pl.ds(base, _CS)],
                        metas[p].at[:, pl.ds(0, _CS)])
        for k in range(3):
            pltpu.async_copy(gt_hbm.at[metas[p].at[k, pl.ds(0, _CS)]],
                             rows[p][k], sems[p][k])

    def drain(p):
        for k in range(3):
            pltpu.make_async_copy(gt_hbm.at[pl.ds(0, _CS)],
                                  rows[p][k], sems[p][k]).wait()

    def compute(ci, p):
        base = base0 + ci * _CS
        mv = metas[p]
        rv = rows[p]

        def point(i, c):
            sp = []
            for k in range(3):
                wv = lax.bitcast_convert_type(mv[3 + k, pl.ds(i, 16)],
                                              jnp.float32)
                sp.append(jnp.full((16,), wv[0], jnp.float32))
            for j in range(_JW):
                sl = pl.ds(j * 16, 16)
                out_v[i, sl] = (rv[0][i, sl] * sp[0] + rv[1][i, sl] * sp[1]
                                + rv[2][i, sl] * sp[2])
            return c

        lax.fori_loop(0, _CS, point, 0)
        pltpu.sync_copy(out_v, out_hbm.at[pl.ds(base, _CS)])

    fire(0, 0)

    def body(t2, carry):
        for p in range(2):
            ci = 2 * t2 + p

            @pl.when(ci + 1 < _NCH)
            def _():
                fire(ci + 1, 1 - p)

            drain(p)
            compute(ci, p)
        return carry

    lax.fori_loop(0, _NCH // 2, body, 0)


def _e_body(interp_ref, uf_ref, w1b_ref, b1_ref, y1_ref, st_ref):
    y = lax.dot_general(uf_ref[...], w1b_ref[...], (((0,), (1,)), ((), ())),
                        preferred_element_type=jnp.float32)
    y = y + interp_ref[...].astype(jnp.float32) + b1_ref[...]
    y1_ref[...] = y
    s = jnp.sum(y, axis=0, keepdims=True)
    s2 = jnp.sum(y * y, axis=0, keepdims=True)

    @pl.when(jnp.logical_and(pl.program_id(0) == 0, pl.program_id(1) == 0))
    def _():
        st_ref[...] = jnp.zeros_like(st_ref)

    st_ref[...] = st_ref[...] + jnp.concatenate(
        [s, s2, jnp.zeros((6, _O1), jnp.float32)], axis=0)


def _f_body(y1_ref, st1_ref, g1_ref, bt1_ref, w2_ref, b2_ref, y2_ref, st_ref):
    st = st1_ref[...]
    mean = st[0:1, :] / _NTOT
    var = st[1:2, :] / _NTOT - mean * mean
    inv = lax.rsqrt(var + 1e-5) * g1_ref[...]
    h = jnp.maximum((y1_ref[...] - mean) * inv + bt1_ref[...], 0.0)
    y2 = lax.dot_general(w2_ref[...], h, (((1,), (1,)), ((), ())),
                         preferred_element_type=jnp.float32)
    y2 = y2 + b2_ref[...]
    y2_ref[...] = y2
    s = jnp.sum(y2, axis=1, keepdims=True)
    s2 = jnp.sum(y2 * y2, axis=1, keepdims=True)

    @pl.when(pl.program_id(0) == 0)
    def _():
        st_ref[...] = jnp.zeros_like(st_ref)

    st_ref[...] = st_ref[...] + jnp.concatenate(
        [s, s2, jnp.zeros((_O2, 6), jnp.float32)], axis=1)


def _g_body(y2_ref, st2_ref, g2_ref, bt2_ref, out_ref):
    st = st2_ref[...]
    mean = st[:, 0:1] / _NTOT
    var = st[:, 1:2] / _NTOT - mean * mean
    inv = lax.rsqrt(var + 1e-5) * g2_ref[...]
    out_ref[...] = jnp.maximum((y2_ref[...] - mean) * inv + bt2_ref[...], 0.0)


def kernel(unknown, known, unknow_feats, known_feats,
           W1, b1, g1, bt1, W2, b2, g2, bt2):
    f32 = jnp.float32
    ut8 = jnp.concatenate(
        [jnp.transpose(unknown, (0, 2, 1)), jnp.zeros((_B, 5, _N), f32)], axis=1)
    kt8 = jnp.concatenate(
        [jnp.transpose(known, (0, 2, 1)), jnp.zeros((_B, 5, _M), f32)], axis=1)
    W1a = W1[:, :_C2]
    W1b = W1[:, _C2:]
    b1r = b1.reshape(1, _O1)
    g1r = g1.reshape(1, _O1)
    bt1r = bt1.reshape(1, _O1)
    b2r = b2.reshape(_O2, 1)
    g2r = g2.reshape(_O2, 1)
    bt2r = bt2.reshape(_O2, 1)

    gt = pl.pallas_call(
        _gt_body,
        grid=(_B,),
        in_specs=[
            pl.BlockSpec((None, _C2, _M), lambda b: (b, 0, 0)),
            pl.BlockSpec((_O1, _C2), lambda b: (0, 0)),
        ],
        out_specs=pl.BlockSpec((_M, _O1), lambda b: (b, 0)),
        out_shape=jax.ShapeDtypeStruct((_B * _M, _O1), f32),
    )(known_feats, W1a)

    nja = _N // _NBLK_A
    meta = pl.pallas_call(
        _knn_body,
        grid=(_B, nja),
        in_specs=[
            pl.BlockSpec((None, 8, _NBLK_A), lambda b, j: (b, 0, j)),
            pl.BlockSpec((None, 8, _M), lambda b, j: (b, 0, 0)),
        ],
        out_specs=pl.BlockSpec((8, _NBLK_A), lambda b, j: (0, b * nja + j)),
        out_shape=jax.ShapeDtypeStruct((8, _NTOT), jnp.int32),
    )(ut8, kt8)

    interp = pl.kernel(
        _interp_body,
        out_type=jax.ShapeDtypeStruct((_NTOT, _O1), f32),
        mesh=plsc.VectorSubcoreMesh(core_axis_name="c", subcore_axis_name="s"),
        scratch_types=[
            pltpu.VMEM((8, _CS + 16), jnp.int32),
            pltpu.VMEM((8, _CS + 16), jnp.int32),
            pltpu.VMEM((_CS, _O1), f32),
            pltpu.VMEM((_CS, _O1), f32),
            pltpu.VMEM((_CS, _O1), f32),
            pltpu.VMEM((_CS, _O1), f32),
            pltpu.VMEM((_CS, _O1), f32),
            pltpu.VMEM((_CS, _O1), f32),
            pltpu.VMEM((_CS, _O1), f32),
            pltpu.SemaphoreType.DMA,
            pltpu.SemaphoreType.DMA,
            pltpu.SemaphoreType.DMA,
            pltpu.SemaphoreType.DMA,
            pltpu.SemaphoreType.DMA,
            pltpu.SemaphoreType.DMA,
        ],
    )(meta, gt)

    nj = _N // _NBLK
    y1, st1 = pl.pallas_call(
        _e_body,
        grid=(_B, nj),
        in_specs=[
            pl.BlockSpec((_NBLK, _O1), lambda b, j: (b * nj + j, 0)),
            pl.BlockSpec((None, _C1, _NBLK), lambda b, j: (b, 0, j)),
            pl.BlockSpec((_O1, _C1), lambda b, j: (0, 0)),
            pl.BlockSpec((1, _O1), lambda b, j: (0, 0)),
        ],
        out_specs=[
            pl.BlockSpec((_NBLK, _O1), lambda b, j: (b * nj + j, 0)),
            pl.BlockSpec((8, _O1), lambda b, j: (0, 0)),
        ],
        out_shape=[
            jax.ShapeDtypeStruct((_NTOT, _O1), f32),
            jax.ShapeDtypeStruct((8, _O1), f32),
        ],
    )(interp, unknow_feats, W1b, b1r)

    nt = _NTOT // _NBLK
    y2, st2 = pl.pallas_call(
        _f_body,
        grid=(nt,),
        in_specs=[
            pl.BlockSpec((_NBLK, _O1), lambda t: (t, 0)),
            pl.BlockSpec((8, _O1), lambda t: (0, 0)),
            pl.BlockSpec((1, _O1), lambda t: (0, 0)),
            pl.BlockSpec((1, _O1), lambda t: (0, 0)),
            pl.BlockSpec((_O2, _O1), lambda t: (0, 0)),
            pl.BlockSpec((_O2, 1), lambda t: (0, 0)),
        ],
        out_specs=[
            pl.BlockSpec((None, _O2, _NBLK), lambda t: (t // nj, 0, t % nj)),
            pl.BlockSpec((_O2, 8), lambda t: (0, 0)),
        ],
        out_shape=[
            jax.ShapeDtypeStruct((_B, _O2, _N), f32),
            jax.ShapeDtypeStruct((_O2, 8), f32),
        ],
    )(y1, st1, g1r, bt1r, W2, b2r)

    out = pl.pallas_call(
        _g_body,
        grid=(_B, nj),
        in_specs=[
            pl.BlockSpec((None, _O2, _NBLK), lambda b, j: (b, 0, j)),
            pl.BlockSpec((_O2, 8), lambda b, j: (0, 0)),
            pl.BlockSpec((_O2, 1), lambda b, j: (0, 0)),
            pl.BlockSpec((_O2, 1), lambda b, j: (0, 0)),
        ],
        out_specs=pl.BlockSpec((None, _O2, _NBLK), lambda b, j: (b, 0, j)),
        out_shape=jax.ShapeDtypeStruct((_B, _O2, _N), f32),
    )(y2, st2, g2r, bt2r)
    return out


# speedup vs baseline: 1.3271x; 1.0320x over previous
import functools

import jax
import jax.numpy as jnp
from jax import lax
from jax.experimental import pallas as pl
from jax.experimental.pallas import tpu as pltpu
from jax.experimental.pallas import tpu_sc as plsc

_B, _N, _M = 16, 4096, 1024
_C1, _C2 = 256, 512
_O1, _O2 = 512, 256
_NTOT = _B * _N
_NBLK_A = 256
_NBLK = 512

_BH = _B // 2
_NTOTH = _BH * _N
_NW = 32
_PPW = _NTOTH // _NW
_CS = 32
_NCH = _PPW // _CS
_JW = _O1 // 16


def _gt_body(kf_ref, w1a_ref, gt_ref):
    gt_ref[...] = lax.dot_general(
        kf_ref[...], w1a_ref[...], (((0,), (1,)), ((), ())),
        preferred_element_type=jnp.float32)


def _knn_body(ut_ref, kt_ref, meta_ref):
    u = ut_ref[...]
    kv = kt_ref[...]
    uu = jnp.sum(u * u, axis=0)[None, :]
    kk = jnp.sum(kv * kv, axis=0)[:, None]
    cross = lax.dot_general(kv, u, (((0,), (0,)), ((), ())),
                            preferred_element_type=jnp.float32)
    d = jnp.maximum(kk + uu - 2.0 * cross, 0.0)
    iota = lax.broadcasted_iota(jnp.int32, (_M, _NBLK_A), 0)
    recips, imins = [], []
    for _ in range(3):
        vmin = jnp.min(d, axis=0, keepdims=True)
        imin = jnp.min(jnp.where(d == vmin, iota, _M), axis=0, keepdims=True)
        recips.append(1.0 / (jnp.sqrt(vmin) + 1e-8))
        imins.append(imin)
        d = jnp.where(iota == imin, jnp.float32(jnp.inf), d)
    norm = recips[0] + recips[1] + recips[2]
    gbase = pl.program_id(0) * _M

    bits = lambda x: lax.bitcast_convert_type(x, jnp.int32)
    meta_ref[...] = jnp.concatenate(
        [imins[0] + gbase, imins[1] + gbase, imins[2] + gbase,
         bits(recips[0] / norm), bits(recips[1] / norm),
         bits(recips[2] / norm),
         jnp.zeros((2, _NBLK_A), jnp.int32)], axis=0)


def _interp_body(meta_hbm, gt_hbm, out_hbm,
                 meta0_v, meta1_v, r00, r01, r02, r10, r11, r12, out_v,
                 s00, s01, s02, s10, s11, s12):
    wid = lax.axis_index("s") * 2 + lax.axis_index("c")
    base0 = wid * _PPW
    metas = (meta0_v, meta1_v)
    rows = ((r00, r01, r02), (r10, r11, r12))
    sems = ((s00, s01, s02), (s10, s11, s12))

    def fire(ci, p):
        base = base0 + ci * _CS
        pltpu.sync_copy(meta_hbm.at[pl.ds(0, 8), pl.ds(base, _CS)],
                        metas[p].at[:, pl.ds(0, _CS)])
        for k in range(3):
            pltpu.async_copy(gt_hbm.at[metas[p].at[k, pl.ds(0, _CS)]],
                             rows[p][k], sems[p][k])

    def drain(p):
        for k in range(3):
            pltpu.make_async_copy(gt_hbm.at[pl.ds(0, _CS)],
                                  rows[p][k], sems[p][k]).wait()

    def compute(ci, p):
        base = base0 + ci * _CS
        mv = metas[p]
        rv = rows[p]

        def point(i, c):
            sp = []
            for k in range(3):
                wv = lax.bitcast_convert_type(mv[3 + k, pl.ds(i, 16)],
                                              jnp.float32)
                sp.append(jnp.full((16,), wv[0], jnp.float32))
            for j in range(_JW):
                sl = pl.ds(j * 16, 16)
                out_v[i, sl] = (rv[0][i, sl] * sp[0] + rv[1][i, sl] * sp[1]
                                + rv[2][i, sl] * sp[2])
            return c

        lax.fori_loop(0, _CS, point, 0)
        pltpu.sync_copy(out_v, out_hbm.at[pl.ds(base, _CS)])

    fire(0, 0)

    def body(t2, carry):
        for p in range(2):
            ci = 2 * t2 + p

            @pl.when(ci + 1 < _NCH)
            def _():
                fire(ci + 1, 1 - p)

            drain(p)
            compute(ci, p)
        return carry

    lax.fori_loop(0, _NCH // 2, body, 0)


def _e_body(interp_ref, uf_ref, w1b_ref, b1_ref, y1_ref, st_ref):
    y = lax.dot_general(uf_ref[...], w1b_ref[...], (((0,), (1,)), ((), ())),
                        preferred_element_type=jnp.float32)
    y = y + interp_ref[...].astype(jnp.float32) + b1_ref[...]
    y1_ref[...] = y
    s = jnp.sum(y, axis=0, keepdims=True)
    s2 = jnp.sum(y * y, axis=0, keepdims=True)

    @pl.when(jnp.logical_and(pl.program_id(0) == 0, pl.program_id(1) == 0))
    def _():
        st_ref[...] = jnp.zeros_like(st_ref)

    st_ref[...] = st_ref[...] + jnp.concatenate(
        [s, s2, jnp.zeros((6, _O1), jnp.float32)], axis=0)


def _f_body(y1_ref, st1_ref, g1_ref, bt1_ref, w2_ref, b2_ref, y2_ref, st_ref):
    st = st1_ref[...]
    mean = st[0:1, :] / _NTOT
    var = st[1:2, :] / _NTOT - mean * mean
    inv = lax.rsqrt(var + 1e-5) * g1_ref[...]
    h = jnp.maximum((y1_ref[...] - mean) * inv + bt1_ref[...], 0.0)
    y2 = lax.dot_general(w2_ref[...], h, (((1,), (1,)), ((), ())),
                         preferred_element_type=jnp.float32)
    y2 = y2 + b2_ref[...]
    y2_ref[...] = y2
    s = jnp.sum(y2, axis=1, keepdims=True)
    s2 = jnp.sum(y2 * y2, axis=1, keepdims=True)

    @pl.when(pl.program_id(0) == 0)
    def _():
        st_ref[...] = jnp.zeros_like(st_ref)

    st_ref[...] = st_ref[...] + jnp.concatenate(
        [s, s2, jnp.zeros((_O2, 6), jnp.float32)], axis=1)


def _g_body(y2_ref, st2_ref, g2_ref, bt2_ref, out_ref):
    st = st2_ref[...]
    mean = st[:, 0:1] / _NTOT
    var = st[:, 1:2] / _NTOT - mean * mean
    inv = lax.rsqrt(var + 1e-5) * g2_ref[...]
    out_ref[...] = jnp.maximum((y2_ref[...] - mean) * inv + bt2_ref[...], 0.0)


def _gt_half(kfh, W1a):
    return pl.pallas_call(
        _gt_body,
        grid=(_BH,),
        in_specs=[
            pl.BlockSpec((None, _C2, _M), lambda b: (b, 0, 0)),
            pl.BlockSpec((_O1, _C2), lambda b: (0, 0)),
        ],
        out_specs=pl.BlockSpec((_M, _O1), lambda b: (b, 0)),
        out_shape=jax.ShapeDtypeStruct((_BH * _M, _O1), jnp.float32),
    )(kfh, W1a)


def _knn_half(ut8h, kt8h):
    nja = _N // _NBLK_A
    return pl.pallas_call(
        _knn_body,
        grid=(_BH, nja),
        in_specs=[
            pl.BlockSpec((None, 8, _NBLK_A), lambda b, j: (b, 0, j)),
            pl.BlockSpec((None, 8, _M), lambda b, j: (b, 0, 0)),
        ],
        out_specs=pl.BlockSpec((8, _NBLK_A), lambda b, j: (0, b * nja + j)),
        out_shape=jax.ShapeDtypeStruct((8, _NTOTH), jnp.int32),
    )(ut8h, kt8h)


def _interp_half(meta_h, gt_h):
    f32 = jnp.float32
    return pl.kernel(
        _interp_body,
        out_type=jax.ShapeDtypeStruct((_NTOTH, _O1), f32),
        mesh=plsc.VectorSubcoreMesh(core_axis_name="c", subcore_axis_name="s"),
        scratch_types=[
            pltpu.VMEM((8, _CS + 16), jnp.int32),
            pltpu.VMEM((8, _CS + 16), jnp.int32),
            pltpu.VMEM((_CS, _O1), f32),
            pltpu.VMEM((_CS, _O1), f32),
            pltpu.VMEM((_CS, _O1), f32),
            pltpu.VMEM((_CS, _O1), f32),
            pltpu.VMEM((_CS, _O1), f32),
            pltpu.VMEM((_CS, _O1), f32),
            pltpu.VMEM((_CS, _O1), f32),
            pltpu.SemaphoreType.DMA,
            pltpu.SemaphoreType.DMA,
            pltpu.SemaphoreType.DMA,
            pltpu.SemaphoreType.DMA,
            pltpu.SemaphoreType.DMA,
            pltpu.SemaphoreType.DMA,
        ],
    )(meta_h, gt_h)


def _e_half(interp_h, ufh, W1b, b1r):
    nj = _N // _NBLK
    return pl.pallas_call(
        _e_body,
        grid=(_BH, nj),
        in_specs=[
            pl.BlockSpec((_NBLK, _O1), lambda b, j: (b * nj + j, 0)),
            pl.BlockSpec((None, _C1, _NBLK), lambda b, j: (b, 0, j)),
            pl.BlockSpec((_O1, _C1), lambda b, j: (0, 0)),
            pl.BlockSpec((1, _O1), lambda b, j: (0, 0)),
        ],
        out_specs=[
            pl.BlockSpec((_NBLK, _O1), lambda b, j: (b * nj + j, 0)),
            pl.BlockSpec((8, _O1), lambda b, j: (0, 0)),
        ],
        out_shape=[
            jax.ShapeDtypeStruct((_NTOTH, _O1), jnp.float32),
            jax.ShapeDtypeStruct((8, _O1), jnp.float32),
        ],
    )(interp_h, ufh, W1b, b1r)


def _f_half(y1_h, st1, g1r, bt1r, W2, b2r):
    nj = _N // _NBLK
    nth = _NTOTH // _NBLK
    return pl.pallas_call(
        _f_body,
        grid=(nth,),
        in_specs=[
            pl.BlockSpec((_NBLK, _O1), lambda t: (t, 0)),
            pl.BlockSpec((8, _O1), lambda t: (0, 0)),
            pl.BlockSpec((1, _O1), lambda t: (0, 0)),
            pl.BlockSpec((1, _O1), lambda t: (0, 0)),
            pl.BlockSpec((_O2, _O1), lambda t: (0, 0)),
            pl.BlockSpec((_O2, 1), lambda t: (0, 0)),
        ],
        out_specs=[
            pl.BlockSpec((None, _O2, _NBLK), lambda t: (t // nj, 0, t % nj)),
            pl.BlockSpec((_O2, 8), lambda t: (0, 0)),
        ],
        out_shape=[
            jax.ShapeDtypeStruct((_BH, _O2, _N), jnp.float32),
            jax.ShapeDtypeStruct((_O2, 8), jnp.float32),
        ],
    )(y1_h, st1, g1r, bt1r, W2, b2r)


def _g_half(y2_h, st2, g2r, bt2r):
    nj = _N // _NBLK
    return pl.pallas_call(
        _g_body,
        grid=(_BH, nj),
        in_specs=[
            pl.BlockSpec((None, _O2, _NBLK), lambda b, j: (b, 0, j)),
            pl.BlockSpec((_O2, 8), lambda b, j: (0, 0)),
            pl.BlockSpec((_O2, 1), lambda b, j: (0, 0)),
            pl.BlockSpec((_O2, 1), lambda b, j: (0, 0)),
        ],
        out_specs=pl.BlockSpec((None, _O2, _NBLK), lambda b, j: (b, 0, j)),
        out_shape=jax.ShapeDtypeStruct((_BH, _O2, _N), jnp.float32),
    )(y2_h, st2, g2r, bt2r)


def kernel(unknown, known, unknow_feats, known_feats,
           W1, b1, g1, bt1, W2, b2, g2, bt2):
    f32 = jnp.float32
    ut8 = jnp.concatenate(
        [jnp.transpose(unknown, (0, 2, 1)), jnp.zeros((_B, 5, _N), f32)], axis=1)
    kt8 = jnp.concatenate(
        [jnp.transpose(known, (0, 2, 1)), jnp.zeros((_B, 5, _M), f32)], axis=1)
    W1a = W1[:, :_C2]
    W1b = W1[:, _C2:]
    b1r = b1.reshape(1, _O1)
    g1r = g1.reshape(1, _O1)
    bt1r = bt1.reshape(1, _O1)
    b2r = b2.reshape(_O2, 1)
    g2r = g2.reshape(_O2, 1)
    bt2r = bt2.reshape(_O2, 1)

    gts, metas, interps, y1s, st1s = [], [], [], [], []
    for h in range(2):
        lo, hi = h * _BH, (h + 1) * _BH
        gts.append(_gt_half(known_feats[lo:hi], W1a))
        metas.append(_knn_half(ut8[lo:hi], kt8[lo:hi]))
    for h in range(2):
        interps.append(_interp_half(metas[h], gts[h]))
    for h in range(2):
        lo, hi = h * _BH, (h + 1) * _BH
        y1_h, st1_h = _e_half(interps[h], unknow_feats[lo:hi], W1b, b1r)
        y1s.append(y1_h)
        st1s.append(st1_h)
    st1 = st1s[0] + st1s[1]
    y2s, st2s = [], []
    for h in range(2):
        y2_h, st2_h = _f_half(y1s[h], st1, g1r, bt1r, W2, b2r)
        y2s.append(y2_h)
        st2s.append(st2_h)
    st2 = st2s[0] + st2s[1]
    outs = [_g_half(y2s[h], st2, g2r, bt2r) for h in range(2)]
    return jnp.concatenate(outs, axis=0)


# speedup vs baseline: 1.7334x; 1.3061x over previous
import functools

import jax
import jax.numpy as jnp
from jax import lax
from jax.experimental import pallas as pl
from jax.experimental.pallas import tpu as pltpu
from jax.experimental.pallas import tpu_sc as plsc

_B, _N, _M = 16, 4096, 1024
_C1, _C2 = 256, 512
_O1, _O2 = 512, 256
_NTOT = _B * _N
_NBLK_A = 256
_NBLK = 512

_BH = _B // 2
_NTOTH = _BH * _N
_NW = 32
_PPW = _NTOTH // _NW
_CS = 32
_NCH = _PPW // _CS
_JW = _O1 // 16


def _gt_body(kf_ref, w1a_ref, gt_ref):
    gt_ref[...] = lax.dot_general(
        kf_ref[...], w1a_ref[...], (((0,), (1,)), ((), ())),
        preferred_element_type=jnp.float32)


def _knn_body(ut_ref, kt_ref, meta_ref):
    u = ut_ref[...]
    kv = kt_ref[...]
    uu = jnp.sum(u * u, axis=0)[None, :]
    kk = jnp.sum(kv * kv, axis=0)[:, None]
    cross = lax.dot_general(kv, u, (((0,), (0,)), ((), ())),
                            preferred_element_type=jnp.float32)
    d = jnp.maximum(kk + uu - 2.0 * cross, 0.0)
    iota = lax.broadcasted_iota(jnp.int32, (_M, _NBLK_A), 0)
    keys = (lax.bitcast_convert_type(d, jnp.int32) & ~jnp.int32(1023)) | iota
    recips, imins = [], []
    for _ in range(3):
        kmin = jnp.min(keys, axis=0, keepdims=True)
        keys = jnp.where(keys == kmin, jnp.int32(0x7FFFFFFF), keys)
        d2 = lax.bitcast_convert_type(kmin & ~jnp.int32(1023), jnp.float32)
        recips.append(1.0 / (jnp.sqrt(d2) + 1e-8))
        imins.append(kmin & jnp.int32(1023))
    norm = recips[0] + recips[1] + recips[2]
    gbase = pl.program_id(0) * _M

    bits = lambda x: lax.bitcast_convert_type(x, jnp.int32)
    meta_ref[...] = jnp.concatenate(
        [imins[0] + gbase, imins[1] + gbase, imins[2] + gbase,
         bits(recips[0] / norm), bits(recips[1] / norm),
         bits(recips[2] / norm),
         jnp.zeros((2, _NBLK_A), jnp.int32)], axis=0)


def _interp_body(meta_hbm, gt_hbm, out_hbm,
                 meta0_v, meta1_v, r00, r01, r02, r10, r11, r12, out_v,
                 s00, s01, s02, s10, s11, s12):
    wid = lax.axis_index("s") * 2 + lax.axis_index("c")
    base0 = wid * _PPW
    metas = (meta0_v, meta1_v)
    rows = ((r00, r01, r02), (r10, r11, r12))
    sems = ((s00, s01, s02), (s10, s11, s12))

    def fire(ci, p):
        base = base0 + ci * _CS
        pltpu.sync_copy(meta_hbm.at[pl.ds(0, 8), pl.ds(base, _CS)],
                        metas[p].at[:, pl.ds(0, _CS)])
        for k in range(3):
            pltpu.async_copy(gt_hbm.at[metas[p].at[k, pl.ds(0, _CS)]],
                             rows[p][k], sems[p][k])

    def drain(p):
        for k in range(3):
            pltpu.make_async_copy(gt_hbm.at[pl.ds(0, _CS)],
                                  rows[p][k], sems[p][k]).wait()

    def compute(ci, p):
        base = base0 + ci * _CS
        mv = metas[p]
        rv = rows[p]

        def point(i, c):
            sp = []
            for k in range(3):
                wv = lax.bitcast_convert_type(mv[3 + k, pl.ds(i, 16)],
                                              jnp.float32)
                sp.append(jnp.full((16,), wv[0], jnp.float32))
            for j in range(_JW):
                sl = pl.ds(j * 16, 16)
                out_v[i, sl] = (rv[0][i, sl] * sp[0] + rv[1][i, sl] * sp[1]
                                + rv[2][i, sl] * sp[2])
            return c

        lax.fori_loop(0, _CS, point, 0)
        pltpu.sync_copy(out_v, out_hbm.at[pl.ds(base, _CS)])

    fire(0, 0)

    def body(t2, carry):
        for p in range(2):
            ci = 2 * t2 + p

            @pl.when(ci + 1 < _NCH)
            def _():
                fire(ci + 1, 1 - p)

            drain(p)
            compute(ci, p)
        return carry

    lax.fori_loop(0, _NCH // 2, body, 0)


def _e_body(interp_ref, uf_ref, w1b_ref, b1_ref, y1_ref, st_ref):
    y = lax.dot_general(uf_ref[...], w1b_ref[...], (((0,), (1,)), ((), ())),
                        preferred_element_type=jnp.float32)
    y = y + interp_ref[...].astype(jnp.float32) + b1_ref[...]
    y1_ref[...] = y.astype(jnp.bfloat16)
    s = jnp.sum(y, axis=0, keepdims=True)
    s2 = jnp.sum(y * y, axis=0, keepdims=True)

    @pl.when(jnp.logical_and(pl.program_id(0) == 0, pl.program_id(1) == 0))
    def _():
        st_ref[...] = jnp.zeros_like(st_ref)

    st_ref[...] = st_ref[...] + jnp.concatenate(
        [s, s2, jnp.zeros((6, _O1), jnp.float32)], axis=0)


def _f_body(y1_ref, st1_ref, g1_ref, bt1_ref, w2_ref, b2_ref, y2_ref, st_ref):
    st = st1_ref[...]
    mean = st[0:1, :] / _NTOT
    var = st[1:2, :] / _NTOT - mean * mean
    inv = lax.rsqrt(var + 1e-5) * g1_ref[...]
    y1 = y1_ref[...].astype(jnp.float32)
    h = jnp.maximum((y1 - mean) * inv + bt1_ref[...], 0.0)
    y2 = lax.dot_general(w2_ref[...], h, (((1,), (1,)), ((), ())),
                         preferred_element_type=jnp.float32)
    y2 = y2 + b2_ref[...]
    y2_ref[...] = y2.astype(jnp.bfloat16)
    s = jnp.sum(y2, axis=1, keepdims=True)
    s2 = jnp.sum(y2 * y2, axis=1, keepdims=True)

    @pl.when(pl.program_id(0) == 0)
    def _():
        st_ref[...] = jnp.zeros_like(st_ref)

    st_ref[...] = st_ref[...] + jnp.concatenate(
        [s, s2, jnp.zeros((_O2, 6), jnp.float32)], axis=1)


def _g_body(y2h0_ref, y2h1_ref, st2_ref, g2_ref, bt2_ref, out_ref):
    st = st2_ref[...]
    mean = st[:, 0:1] / _NTOT
    var = st[:, 1:2] / _NTOT - mean * mean
    inv = lax.rsqrt(var + 1e-5) * g2_ref[...]
    y2 = jnp.where(pl.program_id(0) < _BH, y2h0_ref[...],
                   y2h1_ref[...]).astype(jnp.float32)
    out_ref[...] = jnp.maximum((y2 - mean) * inv + bt2_ref[...], 0.0)


def _gt_half(h, kf, W1a):
    return pl.pallas_call(
        _gt_body,
        grid=(_BH,),
        in_specs=[
            pl.BlockSpec((None, _C2, _M), lambda b: (b + h * _BH, 0, 0)),
            pl.BlockSpec((_O1, _C2), lambda b: (0, 0)),
        ],
        out_specs=pl.BlockSpec((_M, _O1), lambda b: (b, 0)),
        out_shape=jax.ShapeDtypeStruct((_BH * _M, _O1), jnp.float32),
    )(kf, W1a)


def _knn_half(h, ut8, kt8):
    nja = _N // _NBLK_A
    return pl.pallas_call(
        _knn_body,
        grid=(_BH, nja),
        in_specs=[
            pl.BlockSpec((None, 8, _NBLK_A), lambda b, j: (b + h * _BH, 0, j)),
            pl.BlockSpec((None, 8, _M), lambda b, j: (b + h * _BH, 0, 0)),
        ],
        out_specs=pl.BlockSpec((8, _NBLK_A), lambda b, j: (0, b * nja + j)),
        out_shape=jax.ShapeDtypeStruct((8, _NTOTH), jnp.int32),
    )(ut8, kt8)


def _interp_half(meta_h, gt_h):
    f32 = jnp.float32
    return pl.kernel(
        _interp_body,
        out_type=jax.ShapeDtypeStruct((_NTOTH, _O1), f32),
        mesh=plsc.VectorSubcoreMesh(core_axis_name="c", subcore_axis_name="s"),
        scratch_types=[
            pltpu.VMEM((8, _CS + 16), jnp.int32),
            pltpu.VMEM((8, _CS + 16), jnp.int32),
            pltpu.VMEM((_CS, _O1), f32),
            pltpu.VMEM((_CS, _O1), f32),
            pltpu.VMEM((_CS, _O1), f32),
            pltpu.VMEM((_CS, _O1), f32),
            pltpu.VMEM((_CS, _O1), f32),
            pltpu.VMEM((_CS, _O1), f32),
            pltpu.VMEM((_CS, _O1), f32),
            pltpu.SemaphoreType.DMA,
            pltpu.SemaphoreType.DMA,
            pltpu.SemaphoreType.DMA,
            pltpu.SemaphoreType.DMA,
            pltpu.SemaphoreType.DMA,
            pltpu.SemaphoreType.DMA,
        ],
    )(meta_h, gt_h)


def _e_half(h, interp_h, uf, W1b, b1r):
    nj = _N // _NBLK
    return pl.pallas_call(
        _e_body,
        grid=(_BH, nj),
        in_specs=[
            pl.BlockSpec((_NBLK, _O1), lambda b, j: (b * nj + j, 0)),
            pl.BlockSpec((None, _C1, _NBLK), lambda b, j: (b + h * _BH, 0, j)),
            pl.BlockSpec((_O1, _C1), lambda b, j: (0, 0)),
            pl.BlockSpec((1, _O1), lambda b, j: (0, 0)),
        ],
        out_specs=[
            pl.BlockSpec((_NBLK, _O1), lambda b, j: (b * nj + j, 0)),
            pl.BlockSpec((8, _O1), lambda b, j: (0, 0)),
        ],
        out_shape=[
            jax.ShapeDtypeStruct((_NTOTH, _O1), jnp.bfloat16),
            jax.ShapeDtypeStruct((8, _O1), jnp.float32),
        ],
    )(interp_h, uf, W1b, b1r)


def _f_half(y1_h, st1, g1r, bt1r, W2, b2r):
    nj = _N // _NBLK
    nth = _NTOTH // _NBLK
    return pl.pallas_call(
        _f_body,
        grid=(nth,),
        in_specs=[
            pl.BlockSpec((_NBLK, _O1), lambda t: (t, 0)),
            pl.BlockSpec((8, _O1), lambda t: (0, 0)),
            pl.BlockSpec((1, _O1), lambda t: (0, 0)),
            pl.BlockSpec((1, _O1), lambda t: (0, 0)),
            pl.BlockSpec((_O2, _O1), lambda t: (0, 0)),
            pl.BlockSpec((_O2, 1), lambda t: (0, 0)),
        ],
        out_specs=[
            pl.BlockSpec((None, _O2, _NBLK), lambda t: (t // nj, 0, t % nj)),
            pl.BlockSpec((_O2, 8), lambda t: (0, 0)),
        ],
        out_shape=[
            jax.ShapeDtypeStruct((_BH, _O2, _N), jnp.bfloat16),
            jax.ShapeDtypeStruct((_O2, 8), jnp.float32),
        ],
    )(y1_h, st1, g1r, bt1r, W2, b2r)


def _g_full(y2_h0, y2_h1, st2, g2r, bt2r):
    nj = _N // _NBLK
    return pl.pallas_call(
        _g_body,
        grid=(_B, nj),
        in_specs=[
            pl.BlockSpec((None, _O2, _NBLK),
                         lambda b, j: (jnp.minimum(b, _BH - 1), 0, j)),
            pl.BlockSpec((None, _O2, _NBLK),
                         lambda b, j: (jnp.maximum(b - _BH, 0), 0, j)),
            pl.BlockSpec((_O2, 8), lambda b, j: (0, 0)),
            pl.BlockSpec((_O2, 1), lambda b, j: (0, 0)),
            pl.BlockSpec((_O2, 1), lambda b, j: (0, 0)),
        ],
        out_specs=pl.BlockSpec((None, _O2, _NBLK), lambda b, j: (b, 0, j)),
        out_shape=jax.ShapeDtypeStruct((_B, _O2, _N), jnp.float32),
    )(y2_h0, y2_h1, st2, g2r, bt2r)


def kernel(unknown, known, unknow_feats, known_feats,
           W1, b1, g1, bt1, W2, b2, g2, bt2):
    f32 = jnp.float32
    ut8 = jnp.concatenate(
        [jnp.transpose(unknown, (0, 2, 1)), jnp.zeros((_B, 5, _N), f32)], axis=1)
    kt8 = jnp.concatenate(
        [jnp.transpose(known, (0, 2, 1)), jnp.zeros((_B, 5, _M), f32)], axis=1)
    W1a = W1[:, :_C2]
    W1b = W1[:, _C2:]
    b1r = b1.reshape(1, _O1)
    g1r = g1.reshape(1, _O1)
    bt1r = bt1.reshape(1, _O1)
    b2r = b2.reshape(_O2, 1)
    g2r = g2.reshape(_O2, 1)
    bt2r = bt2.reshape(_O2, 1)

    gts, metas, interps, y1s, st1s = [], [], [], [], []
    for h in range(2):
        gts.append(_gt_half(h, known_feats, W1a))
        metas.append(_knn_half(h, ut8, kt8))
    for h in range(2):
        interps.append(_interp_half(metas[h], gts[h]))
    for h in range(2):
        y1_h, st1_h = _e_half(h, interps[h], unknow_feats, W1b, b1r)
        y1s.append(y1_h)
        st1s.append(st1_h)
    st1 = st1s[0] + st1s[1]
    y2s, st2s = [], []
    for h in range(2):
        y2_h, st2_h = _f_half(y1s[h], st1, g1r, bt1r, W2, b2r)
        y2s.append(y2_h)
        st2s.append(st2_h)
    st2 = st2s[0] + st2s[1]
    return _g_full(y2s[0], y2s[1], st2, g2r, bt2r)


# speedup vs baseline: 1.7536x; 1.0117x over previous
import functools

import jax
import jax.numpy as jnp
from jax import lax
from jax.experimental import pallas as pl
from jax.experimental.pallas import tpu as pltpu
from jax.experimental.pallas import tpu_sc as plsc

_B, _N, _M = 16, 4096, 1024
_C1, _C2 = 256, 512
_O1, _O2 = 512, 256
_NTOT = _B * _N
_NBLK_A = 256
_NBLK = 512

_BH = _B // 2
_NTOTH = _BH * _N
_NW = 32
_PPW = _NTOTH // _NW
_CS = 32
_NCH = _PPW // _CS
_JW = _O1 // 16


def _gt_body(kf_ref, w1a_ref, gt_ref):
    gt_ref[...] = lax.dot_general(
        kf_ref[...], w1a_ref[...], (((0,), (1,)), ((), ())),
        preferred_element_type=jnp.float32)


def _knn_body(ut_ref, kt_ref, meta_ref):
    u = ut_ref[...]
    kv = kt_ref[...]
    uu = jnp.sum(u * u, axis=0)[None, :]
    kk = jnp.sum(kv * kv, axis=0)[:, None]
    cross = lax.dot_general(kv, u, (((0,), (0,)), ((), ())),
                            preferred_element_type=jnp.float32)
    d = jnp.maximum(kk + uu - 2.0 * cross, 0.0)
    iota = lax.broadcasted_iota(jnp.int32, (_M, _NBLK_A), 0)
    keys = (lax.bitcast_convert_type(d, jnp.int32) & ~jnp.int32(1023)) | iota
    recips, imins = [], []
    for _ in range(3):
        kmin = jnp.min(keys, axis=0, keepdims=True)
        keys = jnp.where(keys == kmin, jnp.int32(0x7FFFFFFF), keys)
        d2 = lax.bitcast_convert_type(kmin & ~jnp.int32(1023), jnp.float32)
        recips.append(1.0 / (jnp.sqrt(d2) + 1e-8))
        imins.append(kmin & jnp.int32(1023))
    norm = recips[0] + recips[1] + recips[2]
    gbase = pl.program_id(0) * _M

    bits = lambda x: lax.bitcast_convert_type(x, jnp.int32)
    meta_ref[...] = jnp.concatenate(
        [imins[0] + gbase, imins[1] + gbase, imins[2] + gbase,
         bits(recips[0] / norm), bits(recips[1] / norm),
         bits(recips[2] / norm),
         jnp.zeros((2, _NBLK_A), jnp.int32)], axis=0)


def _interp_body(meta_hbm, gt_hbm, out_hbm,
                 meta0_v, meta1_v,
                 r00, r01, r02, r10, r11, r12, out_v,
                 s00, s01, s02, s10, s11, s12):
    wid = lax.axis_index("s") * 2 + lax.axis_index("c")
    base0 = wid * _PPW
    metas = (meta0_v, meta1_v)
    rows = ((r00, r01, r02), (r10, r11, r12))
    sems = ((s00, s01, s02), (s10, s11, s12))

    def fire(ci, p):
        base = base0 + ci * _CS
        pltpu.sync_copy(meta_hbm.at[pl.ds(0, 8), pl.ds(base, _CS)],
                        metas[p].at[:, pl.ds(0, _CS)])
        for k in range(3):
            pltpu.async_copy(gt_hbm.at[metas[p].at[k, pl.ds(0, _CS)]],
                             rows[p][k], sems[p][k])

    def drain(p):
        for k in range(3):
            pltpu.make_async_copy(gt_hbm.at[pl.ds(0, _CS)],
                                  rows[p][k], sems[p][k]).wait()

    def compute(ci, p):
        base = base0 + ci * _CS
        mv = metas[p]
        rv = rows[p]

        def point(i, c):
            sp = []
            for k in range(3):
                wv = lax.bitcast_convert_type(mv[3 + k, pl.ds(i, 16)],
                                              jnp.float32)
                sp.append(jnp.full((16,), wv[0], jnp.float32))
            for j in range(_JW):
                sl = pl.ds(j * 16, 16)
                out_v[i, sl] = (rv[0][i, sl] * sp[0] + rv[1][i, sl] * sp[1]
                                + rv[2][i, sl] * sp[2])
            return c

        lax.fori_loop(0, _CS, point, 0)
        pltpu.sync_copy(out_v, out_hbm.at[pl.ds(base, _CS)])

    fire(0, 0)

    def body(t2, carry):
        for p in range(2):
            ci = 2 * t2 + p

            @pl.when(ci + 1 < _NCH)
            def _():
                fire(ci + 1, 1 - p)

            drain(p)
            compute(ci, p)
        return carry

    lax.fori_loop(0, _NCH // 2, body, 0)


def _e_body(interp_ref, uf_ref, w1b_ref, b1_ref, y1_ref, st_ref):
    y = lax.dot_general(uf_ref[...], w1b_ref[...], (((0,), (1,)), ((), ())),
                        preferred_element_type=jnp.float32)
    y = y + interp_ref[...] + b1_ref[...]
    y1_ref[...] = y.astype(jnp.bfloat16)
    s = jnp.sum(y, axis=0, keepdims=True)
    s2 = jnp.sum(y * y, axis=0, keepdims=True)

    @pl.when(jnp.logical_and(pl.program_id(0) == 0, pl.program_id(1) == 0))
    def _():
        st_ref[...] = jnp.zeros_like(st_ref)

    st_ref[...] = st_ref[...] + jnp.concatenate(
        [s, s2, jnp.zeros((6, _O1), jnp.float32)], axis=0)


def _f_body(y1_ref, st1_ref, g1_ref, bt1_ref, w2_ref, b2_ref, y2_ref, st_ref):
    st = st1_ref[...]
    mean = st[0:1, :] / _NTOT
    var = st[1:2, :] / _NTOT - mean * mean
    inv = lax.rsqrt(var + 1e-5) * g1_ref[...]
    y1 = y1_ref[...].astype(jnp.float32)
    h = jnp.maximum((y1 - mean) * inv + bt1_ref[...], 0.0)
    y2 = lax.dot_general(w2_ref[...], h, (((1,), (1,)), ((), ())),
                         preferred_element_type=jnp.float32)
    y2 = y2 + b2_ref[...]
    y2_ref[...] = y2.astype(jnp.bfloat16)
    s = jnp.sum(y2, axis=1, keepdims=True)
    s2 = jnp.sum(y2 * y2, axis=1, keepdims=True)

    @pl.when(pl.program_id(0) == 0)
    def _():
        st_ref[...] = jnp.zeros_like(st_ref)

    st_ref[...] = st_ref[...] + jnp.concatenate(
        [s, s2, jnp.zeros((_O2, 6), jnp.float32)], axis=1)


def _g_body(y2_ref, st2_ref, g2_ref, bt2_ref, out_ref):
    st = st2_ref[...]
    mean = st[:, 0:1] / _NTOT
    var = st[:, 1:2] / _NTOT - mean * mean
    inv = lax.rsqrt(var + 1e-5) * g2_ref[...]
    y2 = y2_ref[...].astype(jnp.float32)
    out_ref[...] = jnp.maximum((y2 - mean) * inv + bt2_ref[...], 0.0)


def _gt_half(h, kf, W1a):
    return pl.pallas_call(
        _gt_body,
        grid=(_BH,),
        in_specs=[
            pl.BlockSpec((None, _C2, _M), lambda b: (b + h * _BH, 0, 0)),
            pl.BlockSpec((_O1, _C2), lambda b: (0, 0)),
        ],
        out_specs=pl.BlockSpec((_M, _O1), lambda b: (b, 0)),
        out_shape=jax.ShapeDtypeStruct((_BH * _M, _O1), jnp.float32),
    )(kf, W1a)


def _knn_half(h, ut8, kt8):
    nja = _N // _NBLK_A
    return pl.pallas_call(
        _knn_body,
        grid=(_BH, nja),
        in_specs=[
            pl.BlockSpec((None, 8, _NBLK_A), lambda b, j: (b + h * _BH, 0, j)),
            pl.BlockSpec((None, 8, _M), lambda b, j: (b + h * _BH, 0, 0)),
        ],
        out_specs=pl.BlockSpec((8, _NBLK_A), lambda b, j: (0, b * nja + j)),
        out_shape=jax.ShapeDtypeStruct((8, _NTOTH), jnp.int32),
    )(ut8, kt8)


def _interp_half(meta_h, gt_h):
    f32 = jnp.float32
    return pl.kernel(
        _interp_body,
        out_type=jax.ShapeDtypeStruct((_NTOTH, _O1), f32),
        mesh=plsc.VectorSubcoreMesh(core_axis_name="c", subcore_axis_name="s"),
        scratch_types=[
            pltpu.VMEM((8, _CS + 16), jnp.int32),
            pltpu.VMEM((8, _CS + 16), jnp.int32),
            pltpu.VMEM((_CS, _O1), f32),
            pltpu.VMEM((_CS, _O1), f32),
            pltpu.VMEM((_CS, _O1), f32),
            pltpu.VMEM((_CS, _O1), f32),
            pltpu.VMEM((_CS, _O1), f32),
            pltpu.VMEM((_CS, _O1), f32),
            pltpu.VMEM((_CS, _O1), f32),
            pltpu.SemaphoreType.DMA,
            pltpu.SemaphoreType.DMA,
            pltpu.SemaphoreType.DMA,
            pltpu.SemaphoreType.DMA,
            pltpu.SemaphoreType.DMA,
            pltpu.SemaphoreType.DMA,
        ],
    )(meta_h, gt_h)


def _e_half(h, interp_h, uf, W1b, b1r):
    nj = _N // _NBLK
    return pl.pallas_call(
        _e_body,
        grid=(_BH, nj),
        in_specs=[
            pl.BlockSpec((_NBLK, _O1), lambda b, j: (b * nj + j, 0)),
            pl.BlockSpec((None, _C1, _NBLK), lambda b, j: (b + h * _BH, 0, j)),
            pl.BlockSpec((_O1, _C1), lambda b, j: (0, 0)),
            pl.BlockSpec((1, _O1), lambda b, j: (0, 0)),
        ],
        out_specs=[
            pl.BlockSpec((_NBLK, _O1), lambda b, j: (b * nj + j, 0)),
            pl.BlockSpec((8, _O1), lambda b, j: (0, 0)),
        ],
        out_shape=[
            jax.ShapeDtypeStruct((_NTOTH, _O1), jnp.bfloat16),
            jax.ShapeDtypeStruct((8, _O1), jnp.float32),
        ],
    )(interp_h, uf, W1b, b1r)


def _f_half(h, y1_h, st1, g1r, bt1r, W2, b2r, y2_prev=None):
    nj = _N // _NBLK
    nth = _NTOTH // _NBLK
    in_specs = [
        pl.BlockSpec((_NBLK, _O1), lambda t: (t, 0)),
        pl.BlockSpec((8, _O1), lambda t: (0, 0)),
        pl.BlockSpec((1, _O1), lambda t: (0, 0)),
        pl.BlockSpec((1, _O1), lambda t: (0, 0)),
        pl.BlockSpec((_O2, _O1), lambda t: (0, 0)),
        pl.BlockSpec((_O2, 1), lambda t: (0, 0)),
    ]
    args = [y1_h, st1, g1r, bt1r, W2, b2r]
    aliases = {}
    body = _f_body
    if y2_prev is not None:
        in_specs.append(pl.BlockSpec((None, 8, 128), lambda t: (0, 0, 0)))
        args.append(y2_prev)
        aliases = {6: 0}
        body = lambda a, b, c, d, e, f, _unused, y2, st: _f_body(
            a, b, c, d, e, f, y2, st)
    return pl.pallas_call(
        body,
        grid=(nth,),
        in_specs=in_specs,
        out_specs=[
            pl.BlockSpec((None, _O2, _NBLK),
                         lambda t, _h=h: (t // nj + _h * _BH, 0, t % nj)),
            pl.BlockSpec((_O2, 8), lambda t: (0, 0)),
        ],
        out_shape=[
            jax.ShapeDtypeStruct((_B, _O2, _N), jnp.bfloat16),
            jax.ShapeDtypeStruct((_O2, 8), jnp.float32),
        ],
        input_output_aliases=aliases,
    )(*args)


def _g_full(y2, st2, g2r, bt2r):
    nj = _N // _NBLK
    return pl.pallas_call(
        _g_body,
        grid=(_B, nj),
        in_specs=[
            pl.BlockSpec((None, _O2, _NBLK), lambda b, j: (b, 0, j)),
            pl.BlockSpec((_O2, 8), lambda b, j: (0, 0)),
            pl.BlockSpec((_O2, 1), lambda b, j: (0, 0)),
            pl.BlockSpec((_O2, 1), lambda b, j: (0, 0)),
        ],
        out_specs=pl.BlockSpec((None, _O2, _NBLK), lambda b, j: (b, 0, j)),
        out_shape=jax.ShapeDtypeStruct((_B, _O2, _N), jnp.float32),
    )(y2, st2, g2r, bt2r)


def kernel(unknown, known, unknow_feats, known_feats,
           W1, b1, g1, bt1, W2, b2, g2, bt2):
    f32 = jnp.float32
    ut8 = jnp.concatenate(
        [jnp.transpose(unknown, (0, 2, 1)), jnp.zeros((_B, 5, _N), f32)], axis=1)
    kt8 = jnp.concatenate(
        [jnp.transpose(known, (0, 2, 1)), jnp.zeros((_B, 5, _M), f32)], axis=1)
    W1a = W1[:, :_C2]
    W1b = W1[:, _C2:]
    b1r = b1.reshape(1, _O1)
    g1r = g1.reshape(1, _O1)
    bt1r = bt1.reshape(1, _O1)
    b2r = b2.reshape(_O2, 1)
    g2r = g2.reshape(_O2, 1)
    bt2r = bt2.reshape(_O2, 1)

    gts, metas, interps, y1s, st1s = [], [], [], [], []
    for h in range(2):
        gts.append(_gt_half(h, known_feats, W1a))
        metas.append(_knn_half(h, ut8, kt8))
    for h in range(2):
        interps.append(_interp_half(metas[h], gts[h]))
    for h in range(2):
        y1_h, st1_h = _e_half(h, interps[h], unknow_feats, W1b, b1r)
        y1s.append(y1_h)
        st1s.append(st1_h)
    st1 = st1s[0] + st1s[1]
    y2_0, st2_0 = _f_half(0, y1s[0], st1, g1r, bt1r, W2, b2r)
    y2, st2_1 = _f_half(1, y1s[1], st1, g1r, bt1r, W2, b2r, y2_prev=y2_0)
    st2 = st2_0 + st2_1
    return _g_full(y2, st2, g2r, bt2r)


# speedup vs baseline: 1.8588x; 1.0600x over previous
import functools

import jax
import jax.numpy as jnp
from jax import lax
from jax.experimental import pallas as pl
from jax.experimental.pallas import tpu as pltpu
from jax.experimental.pallas import tpu_sc as plsc

_B, _N, _M = 16, 4096, 1024
_C1, _C2 = 256, 512
_O1, _O2 = 512, 256
_NTOT = _B * _N
_NBLK_A = 512
_NBLK = 512

_BH = _B // 2
_NTOTH = _BH * _N
_NW = 32
_PPW = _NTOTH // _NW
_CS = 32
_NCH = _PPW // _CS
_JW = _O1 // 16


def _gt_body(kf_ref, w1a_ref, gt_ref):
    gt_ref[...] = lax.dot_general(
        kf_ref[...], w1a_ref[...], (((0,), (1,)), ((), ())),
        preferred_element_type=jnp.float32)


def _knn_body(ut_ref, kt_ref, meta_ref):
    u = ut_ref[...]
    kv = kt_ref[...]
    uu = jnp.sum(u * u, axis=0)[None, :]
    kk = jnp.sum(kv * kv, axis=0)[:, None]
    cross = lax.dot_general(kv, u, (((0,), (0,)), ((), ())),
                            preferred_element_type=jnp.float32)
    d = jnp.maximum(kk + uu - 2.0 * cross, 0.0)
    iota = lax.broadcasted_iota(jnp.int32, (_M, _NBLK_A), 0)
    keys = (lax.bitcast_convert_type(d, jnp.int32) & ~jnp.int32(1023)) | iota
    recips, imins = [], []
    for t in range(3):
        kmin = jnp.min(keys, axis=0, keepdims=True)
        if t < 2:
            keys = jnp.where(keys == kmin, jnp.int32(0x7FFFFFFF), keys)
        d2 = lax.bitcast_convert_type(kmin & ~jnp.int32(1023), jnp.float32)
        recips.append(1.0 / (jnp.sqrt(d2) + 1e-8))
        imins.append(kmin & jnp.int32(1023))
    norm = recips[0] + recips[1] + recips[2]
    gbase = pl.program_id(0) * _M

    bits = lambda x: lax.bitcast_convert_type(x, jnp.int32)
    meta_ref[...] = jnp.concatenate(
        [imins[0] + gbase, imins[1] + gbase, imins[2] + gbase,
         bits(recips[0] / norm), bits(recips[1] / norm),
         bits(recips[2] / norm),
         jnp.zeros((2, _NBLK_A), jnp.int32)], axis=0)


def _interp_body(meta_hbm, gt_hbm, out_hbm,
                 meta0_v, meta1_v,
                 r00, r01, r02, r10, r11, r12, out_v,
                 s00, s01, s02, s10, s11, s12):
    wid = lax.axis_index("s") * 2 + lax.axis_index("c")
    base0 = wid * _PPW
    metas = (meta0_v, meta1_v)
    rows = ((r00, r01, r02), (r10, r11, r12))
    sems = ((s00, s01, s02), (s10, s11, s12))

    def fire(ci, p):
        base = base0 + ci * _CS
        pltpu.sync_copy(meta_hbm.at[pl.ds(0, 8), pl.ds(base, _CS)],
                        metas[p].at[:, pl.ds(0, _CS)])
        for k in range(3):
            pltpu.async_copy(gt_hbm.at[metas[p].at[k, pl.ds(0, _CS)]],
                             rows[p][k], sems[p][k])

    def drain(p):
        for k in range(3):
            pltpu.make_async_copy(gt_hbm.at[pl.ds(0, _CS)],
                                  rows[p][k], sems[p][k]).wait()

    def compute(ci, p):
        base = base0 + ci * _CS
        mv = metas[p]
        rv = rows[p]

        def point(i, c):
            sp = []
            for k in range(3):
                wv = lax.bitcast_convert_type(mv[3 + k, pl.ds(i, 16)],
                                              jnp.float32)
                sp.append(jnp.full((16,), wv[0], jnp.float32))
            for j in range(_JW):
                sl = pl.ds(j * 16, 16)
                out_v[i, sl] = (rv[0][i, sl] * sp[0] + rv[1][i, sl] * sp[1]
                                + rv[2][i, sl] * sp[2])
            return c

        lax.fori_loop(0, _CS, point, 0)
        pltpu.sync_copy(out_v, out_hbm.at[pl.ds(base, _CS)])

    fire(0, 0)

    def body(t2, carry):
        for p in range(2):
            ci = 2 * t2 + p

            @pl.when(ci + 1 < _NCH)
            def _():
                fire(ci + 1, 1 - p)

            drain(p)
            compute(ci, p)
        return carry

    lax.fori_loop(0, _NCH // 2, body, 0)


def _e_body(interp_ref, uf_ref, w1b_ref, b1_ref, y1_ref, st_ref):
    y = lax.dot_general(uf_ref[...].astype(jnp.bfloat16),
                        w1b_ref[...].astype(jnp.bfloat16),
                        (((0,), (1,)), ((), ())),
                        preferred_element_type=jnp.float32)
    y = y + interp_ref[...] + b1_ref[...]
    y1_ref[...] = y.astype(jnp.bfloat16)
    s = jnp.sum(y, axis=0, keepdims=True)
    s2 = jnp.sum(y * y, axis=0, keepdims=True)

    @pl.when(jnp.logical_and(pl.program_id(0) == 0, pl.program_id(1) == 0))
    def _():
        st_ref[...] = jnp.zeros_like(st_ref)

    st_ref[...] = st_ref[...] + jnp.concatenate(
        [s, s2, jnp.zeros((6, _O1), jnp.float32)], axis=0)


def _f_body(y1_ref, st1_ref, g1_ref, bt1_ref, w2_ref, b2_ref, y2_ref, st_ref):
    st = st1_ref[...]
    mean = st[0:1, :] / _NTOT
    var = st[1:2, :] / _NTOT - mean * mean
    inv = lax.rsqrt(var + 1e-5) * g1_ref[...]
    y1 = y1_ref[...].astype(jnp.float32)
    h = jnp.maximum((y1 - mean) * inv + bt1_ref[...], 0.0)
    y2 = lax.dot_general(w2_ref[...].astype(jnp.bfloat16),
                         h.astype(jnp.bfloat16), (((1,), (1,)), ((), ())),
                         preferred_element_type=jnp.float32)
    y2 = y2 + b2_ref[...]
    y2_ref[...] = y2.astype(jnp.bfloat16)
    s = jnp.sum(y2, axis=1, keepdims=True)
    s2 = jnp.sum(y2 * y2, axis=1, keepdims=True)

    @pl.when(pl.program_id(0) == 0)
    def _():
        st_ref[...] = jnp.zeros_like(st_ref)

    st_ref[...] = st_ref[...] + jnp.concatenate(
        [s, s2, jnp.zeros((_O2, 6), jnp.float32)], axis=1)


def _g_body(y2_ref, st2_ref, g2_ref, bt2_ref, out_ref):
    st = st2_ref[...]
    mean = st[:, 0:1] / _NTOT
    var = st[:, 1:2] / _NTOT - mean * mean
    inv = lax.rsqrt(var + 1e-5) * g2_ref[...]
    y2 = y2_ref[...].astype(jnp.float32)
    out_ref[...] = jnp.maximum((y2 - mean) * inv + bt2_ref[...], 0.0)


def _gt_half(h, kf, W1a):
    return pl.pallas_call(
        _gt_body,
        grid=(_BH,),
        in_specs=[
            pl.BlockSpec((None, _C2, _M), lambda b: (b + h * _BH, 0, 0)),
            pl.BlockSpec((_O1, _C2), lambda b: (0, 0)),
        ],
        out_specs=pl.BlockSpec((_M, _O1), lambda b: (b, 0)),
        out_shape=jax.ShapeDtypeStruct((_BH * _M, _O1), jnp.float32),
    )(kf, W1a)


def _knn_half(h, ut8, kt8):
    nja = _N // _NBLK_A
    return pl.pallas_call(
        _knn_body,
        grid=(_BH, nja),
        in_specs=[
            pl.BlockSpec((None, 8, _NBLK_A), lambda b, j: (b + h * _BH, 0, j)),
            pl.BlockSpec((None, 8, _M), lambda b, j: (b + h * _BH, 0, 0)),
        ],
        out_specs=pl.BlockSpec((8, _NBLK_A), lambda b, j: (0, b * nja + j)),
        out_shape=jax.ShapeDtypeStruct((8, _NTOTH), jnp.int32),
    )(ut8, kt8)


def _interp_half(meta_h, gt_h):
    f32 = jnp.float32
    return pl.kernel(
        _interp_body,
        out_type=jax.ShapeDtypeStruct((_NTOTH, _O1), f32),
        mesh=plsc.VectorSubcoreMesh(core_axis_name="c", subcore_axis_name="s"),
        scratch_types=[
            pltpu.VMEM((8, _CS + 16), jnp.int32),
            pltpu.VMEM((8, _CS + 16), jnp.int32),
            pltpu.VMEM((_CS, _O1), f32),
            pltpu.VMEM((_CS, _O1), f32),
            pltpu.VMEM((_CS, _O1), f32),
            pltpu.VMEM((_CS, _O1), f32),
            pltpu.VMEM((_CS, _O1), f32),
            pltpu.VMEM((_CS, _O1), f32),
            pltpu.VMEM((_CS, _O1), f32),
            pltpu.SemaphoreType.DMA,
            pltpu.SemaphoreType.DMA,
            pltpu.SemaphoreType.DMA,
            pltpu.SemaphoreType.DMA,
            pltpu.SemaphoreType.DMA,
            pltpu.SemaphoreType.DMA,
        ],
    )(meta_h, gt_h)


def _e_half(h, interp_h, uf, W1b, b1r):
    nj = _N // _NBLK
    return pl.pallas_call(
        _e_body,
        grid=(_BH, nj),
        in_specs=[
            pl.BlockSpec((_NBLK, _O1), lambda b, j: (b * nj + j, 0)),
            pl.BlockSpec((None, _C1, _NBLK), lambda b, j: (b + h * _BH, 0, j)),
            pl.BlockSpec((_O1, _C1), lambda b, j: (0, 0)),
            pl.BlockSpec((1, _O1), lambda b, j: (0, 0)),
        ],
        out_specs=[
            pl.BlockSpec((_NBLK, _O1), lambda b, j: (b * nj + j, 0)),
            pl.BlockSpec((8, _O1), lambda b, j: (0, 0)),
        ],
        out_shape=[
            jax.ShapeDtypeStruct((_NTOTH, _O1), jnp.bfloat16),
            jax.ShapeDtypeStruct((8, _O1), jnp.float32),
        ],
    )(interp_h, uf, W1b, b1r)


def _f_half(h, y1_h, st1, g1r, bt1r, W2, b2r, y2_prev=None):
    nj = _N // _NBLK
    nth = _NTOTH // _NBLK
    in_specs = [
        pl.BlockSpec((_NBLK, _O1), lambda t: (t, 0)),
        pl.BlockSpec((8, _O1), lambda t: (0, 0)),
        pl.BlockSpec((1, _O1), lambda t: (0, 0)),
        pl.BlockSpec((1, _O1), lambda t: (0, 0)),
        pl.BlockSpec((_O2, _O1), lambda t: (0, 0)),
        pl.BlockSpec((_O2, 1), lambda t: (0, 0)),
    ]
    args = [y1_h, st1, g1r, bt1r, W2, b2r]
    aliases = {}
    body = _f_body
    if y2_prev is not None:
        in_specs.append(pl.BlockSpec((None, 8, 128), lambda t: (0, 0, 0)))
        args.append(y2_prev)
        aliases = {6: 0}
        body = lambda a, b, c, d, e, f, _unused, y2, st: _f_body(
            a, b, c, d, e, f, y2, st)
    return pl.pallas_call(
        body,
        grid=(nth,),
        in_specs=in_specs,
        out_specs=[
            pl.BlockSpec((None, _O2, _NBLK),
                         lambda t, _h=h: (t // nj + _h * _BH, 0, t % nj)),
            pl.BlockSpec((_O2, 8), lambda t: (0, 0)),
        ],
        out_shape=[
            jax.ShapeDtypeStruct((_B, _O2, _N), jnp.bfloat16),
            jax.ShapeDtypeStruct((_O2, 8), jnp.float32),
        ],
        input_output_aliases=aliases,
    )(*args)


def _g_full(y2, st2, g2r, bt2r):
    nj = _N // _NBLK
    return pl.pallas_call(
        _g_body,
        grid=(_B, nj),
        in_specs=[
            pl.BlockSpec((None, _O2, _NBLK), lambda b, j: (b, 0, j)),
            pl.BlockSpec((_O2, 8), lambda b, j: (0, 0)),
            pl.BlockSpec((_O2, 1), lambda b, j: (0, 0)),
            pl.BlockSpec((_O2, 1), lambda b, j: (0, 0)),
        ],
        out_specs=pl.BlockSpec((None, _O2, _NBLK), lambda b, j: (b, 0, j)),
        out_shape=jax.ShapeDtypeStruct((_B, _O2, _N), jnp.float32),
    )(y2, st2, g2r, bt2r)


def kernel(unknown, known, unknow_feats, known_feats,
           W1, b1, g1, bt1, W2, b2, g2, bt2):
    f32 = jnp.float32
    ut8 = jnp.concatenate(
        [jnp.transpose(unknown, (0, 2, 1)), jnp.zeros((_B, 5, _N), f32)], axis=1)
    kt8 = jnp.concatenate(
        [jnp.transpose(known, (0, 2, 1)), jnp.zeros((_B, 5, _M), f32)], axis=1)
    W1a = W1[:, :_C2]
    W1b = W1[:, _C2:]
    b1r = b1.reshape(1, _O1)
    g1r = g1.reshape(1, _O1)
    bt1r = bt1.reshape(1, _O1)
    b2r = b2.reshape(_O2, 1)
    g2r = g2.reshape(_O2, 1)
    bt2r = bt2.reshape(_O2, 1)

    gts, metas, interps, y1s, st1s = [], [], [], [], []
    for h in range(2):
        gts.append(_gt_half(h, known_feats, W1a))
        metas.append(_knn_half(h, ut8, kt8))
    for h in range(2):
        interps.append(_interp_half(metas[h], gts[h]))
    for h in range(2):
        y1_h, st1_h = _e_half(h, interps[h], unknow_feats, W1b, b1r)
        y1s.append(y1_h)
        st1s.append(st1_h)
    st1 = st1s[0] + st1s[1]
    y2_0, st2_0 = _f_half(0, y1s[0], st1, g1r, bt1r, W2, b2r)
    y2, st2_1 = _f_half(1, y1s[1], st1, g1r, bt1r, W2, b2r, y2_prev=y2_0)
    st2 = st2_0 + st2_1
    return _g_full(y2, st2, g2r, bt2r)
